# baseline (speedup 1.0000x reference)
import functools

import jax
import jax.numpy as jnp
from jax import lax
from jax.experimental import pallas as pl
from jax.experimental.pallas import tpu as pltpu
from jax.experimental.pallas import tpu_sc as plsc

N = 10000
NP = 10240
E = 320000
D_IN = 128
D_H = 64
OUT_DIM = 14
SEQ_OUT = 12
NGRAPH = 32

EPC = E // 2
CB = 2000
CBR = 25
CBW = 80
NCH = EPC // CB
WCH = 5
CPT = 4
SEG = NP // 16

_f32 = jnp.float32


def _lrelu(x):
    return jnp.maximum(x, 0.2 * x)


def _tc1_body(xt_ref, w1t_ref, a1s_ref, a1d_ref, ht_ref, as_ref, ad_ref, mb_ref):
    ht = jnp.dot(w1t_ref[...], xt_ref[...], preferred_element_type=_f32)
    ht_ref[...] = ht
    a_s = jnp.dot(a1s_ref[...], ht, preferred_element_type=_f32)
    as_ref[...] = a_s
    ad_ref[...] = jnp.dot(a1d_ref[...], ht, preferred_element_type=_f32)
    mb_ref[...] = jnp.full((128,), jnp.max(a_s), _f32)


def _agg(osum_ref, den_ref, ht_ref, as_ref, ad_ref, b_ref):
    a_s = as_ref[...]
    a_d = ad_ref[...]
    m = jnp.max(a_s)
    ws = jnp.exp(_lrelu(a_s + a_d) - _lrelu(m + a_d))
    den = den_ref[0] + den_ref[1] + ws
    osum = osum_ref[0] + osum_ref[1] + ws[None, :] * ht_ref[...]
    return jnp.maximum(osum / den[None, :] + b_ref[...][:, None], 0.0)


def _tc2_body(osum_ref, den_ref, ht_ref, as_ref, ad_ref, b_ref, w2t_ref,
              a2s_ref, a2d_ref, h2t_ref, as2_ref, ad2_ref, mb_ref):
    hrt = _agg(osum_ref, den_ref, ht_ref, as_ref, ad_ref, b_ref)
    h2t = jnp.dot(w2t_ref[...], hrt, preferred_element_type=_f32)
    h2t_ref[...] = h2t
    a_s2 = jnp.dot(a2s_ref[...], h2t, preferred_element_type=_f32)
    as2_ref[...] = a_s2
    ad2_ref[...] = jnp.dot(a2d_ref[...], h2t, preferred_element_type=_f32)
    mb_ref[...] = jnp.full((128,), jnp.max(a_s2), _f32)


def _tc3_body(osum_ref, den_ref, ht_ref, as_ref, ad_ref, b_ref, batch_ref,
              wmu_ref, bmu_ref, wsg_ref, bsg_ref, mu_ref, sg_ref):
    hrt = _agg(osum_ref, den_ref, ht_ref, as_ref, ad_ref, b_ref)
    gids = lax.broadcasted_iota(jnp.int32, (NP, NGRAPH), 1)
    oh = (batch_ref[...][:, None] == gids).astype(_f32)
    gt = jnp.dot(hrt, oh, preferred_element_type=_f32)
    cnt = jnp.sum(oh, axis=0)
    g = jnp.transpose(gt / jnp.maximum(cnt, 1.0)[None, :])
    mu_ref[...] = jnp.dot(g, wmu_ref[...], preferred_element_type=_f32) + bmu_ref[...]
    z = jnp.dot(g, wsg_ref[...], preferred_element_type=_f32) + bsg_ref[...]
    sg_ref[...] = jnp.maximum(z, 0.0) + jnp.log(1.0 + jnp.exp(-jnp.abs(z)))


def _sc_edge_body(ht_hbm, as_hbm, ad_hbm, mb_hbm, src_hbm, dst_hbm,
                  osum_hbm, den_hbm,
                  as_v, ad_v, m_v, hcols, acc, tmp_v,
                  srcb0, srcb1, dstb0, dstb1, wb0, wb1,
                  w_sh, den_sh,
                  sg0, sg1, sd0, sd1, sw0, sw1):
    core = lax.axis_index("c")
    sid = lax.axis_index("s")
    base = sid * SEG

    srcb = [srcb0, srcb1]
    dstb = [dstb0, dstb1]
    wb = [wb0, wb1]
    sg = [sg0, sg1]
    sd = [sd0, sd1]
    sw = [sw0, sw1]

    pltpu.sync_copy(as_hbm, as_v)
    pltpu.sync_copy(ad_hbm, ad_v)
    pltpu.sync_copy(mb_hbm, m_v)
    pltpu.sync_copy(ht_hbm.at[pl.ds(sid * CPT, CPT)], hcols)
    mv = m_v[pl.ds(0, 16)]

    zero16 = jnp.zeros((16,), _f32)

    def _ztmp(i, _):
        tmp_v[pl.ds(i * 16, 16)] = zero16
        return 0
    lax.fori_loop(0, SEG // 16, _ztmp, 0)
    pltpu.sync_copy(tmp_v, den_sh.at[pl.ds(base, SEG)])

    def _zacc(i, _):
        for q in range(CPT):
            acc[q, pl.ds(i * 16, 16)] = zero16
        return 0
    lax.fori_loop(0, NP // 16, _zacc, 0)

    plsc.subcore_barrier()

    def _wchunk(t, p):
        k = sid * WCH + t
        pltpu.make_async_copy(src_hbm.at[core, k], srcb[p], sg[p]).wait()
        pltpu.make_async_copy(dst_hbm.at[core, k], dstb[p], sd[p]).wait()

        def _wvec(j, _):
            for u in range(CBW // 16):
                s_vec = srcb[p][j, pl.ds(u * 16, 16)]
                d_vec = dstb[p][j, pl.ds(u * 16, 16)]
                a_sv = plsc.load_gather(as_v, [s_vec])
                a_dv = plsc.load_gather(ad_v, [d_vec])
                t1 = a_sv + a_dv
                t2 = mv + a_dv
                w = jnp.exp(jnp.maximum(t1, 0.2 * t1) - jnp.maximum(t2, 0.2 * t2))
                wb[p][j, pl.ds(u * 16, 16)] = w
            return 0
        lax.fori_loop(0, CBR, _wvec, 0)

        pltpu.async_copy(wb[p], w_sh.at[k], sw[p])
        for j in range(CBR):
            pltpu.async_copy(wb[p].at[j], den_sh.at[dstb[p].at[j]], sw[p], add=True)
        pltpu.make_async_copy(wb[p], w_sh.at[k], sw[p]).wait()
        for j in range(CBR):
            pltpu.make_async_copy(wb[p].at[j], den_sh.at[dstb[p].at[j]], sw[p]).wait()

    for t in range(WCH):
        p = t % 2
        k = sid * WCH + t
        if t == 0:
            pltpu.async_copy(src_hbm.at[core, k], srcb[p], sg[p])
            pltpu.async_copy(dst_hbm.at[core, k], dstb[p], sd[p])
        if t + 1 < WCH:
            kn = k + 1
            pn = (t + 1) % 2
            pltpu.async_copy(src_hbm.at[core, kn], srcb[pn], sg[pn])
            pltpu.async_copy(dst_hbm.at[core, kn], dstb[pn], sd[pn])
        _wchunk(t, p)

    plsc.subcore_barrier()

    pltpu.sync_copy(den_sh.at[pl.ds(base, SEG)], tmp_v)
    pltpu.sync_copy(tmp_v, den_hbm.at[core, pl.ds(base, SEG)])

    def _astart(k, p):
        pltpu.async_copy(src_hbm.at[core, k], srcb[p], sg[p])
        pltpu.async_copy(dst_hbm.at[core, k], dstb[p], sd[p])
        pltpu.async_copy(w_sh.at[k], wb[p], sw[p])

    def _await(k, p):
        pltpu.make_async_copy(src_hbm.at[core, k], srcb[p], sg[p]).wait()
        pltpu.make_async_copy(dst_hbm.at[core, k], dstb[p], sd[p]).wait()
        pltpu.make_async_copy(w_sh.at[k], wb[p], sw[p]).wait()

    def _achunk(p):
        def _avec(j, _):
            for u in range(CBW // 16):
                s_vec = srcb[p][j, pl.ds(u * 16, 16)]
                d_vec = dstb[p][j, pl.ds(u * 16, 16)]
                w_vec = wb[p][j, pl.ds(u * 16, 16)]
                for q in range(CPT):
                    hv = plsc.load_gather(hcols.at[q], [s_vec])
                    plsc.addupdate_scatter(acc.at[q], [d_vec], w_vec * hv)
            return 0
        lax.fori_loop(0, CBR, _avec, 0)

    _astart(0, 0)

    def _aouter(g, _):
        for p in range(2):
            k = 2 * g + p
            _await(k, p)

            @pl.when(k + 1 < NCH)
            def _():
                _astart(k + 1, (p + 1) % 2)
            _achunk(p)
        return 0
    lax.fori_loop(0, NCH // 2, _aouter, 0)

    pltpu.sync_copy(acc, osum_hbm.at[core, pl.ds(sid * CPT, CPT)])


_sc_edge = functools.partial(
    pl.kernel,
    out_type=[jax.ShapeDtypeStruct((2, D_H, NP), _f32),
              jax.ShapeDtypeStruct((2, NP), _f32)],
    mesh=plsc.VectorSubcoreMesh(core_axis_name="c", subcore_axis_name="s"),
    compiler_params=pltpu.CompilerParams(needs_layout_passes=False,
                                         use_tc_tiling_on_sc=False),
    scratch_types=[
        pltpu.VMEM((NP,), _f32),
        pltpu.VMEM((NP,), _f32),
        pltpu.VMEM((128,), _f32),
        pltpu.VMEM((CPT, NP), _f32),
        pltpu.VMEM((CPT, NP), _f32),
        pltpu.VMEM((SEG,), _f32),
        pltpu.VMEM((CBR, CBW), jnp.int32),
        pltpu.VMEM((CBR, CBW), jnp.int32),
        pltpu.VMEM((CBR, CBW), jnp.int32),
        pltpu.VMEM((CBR, CBW), jnp.int32),
        pltpu.VMEM((CBR, CBW), _f32),
        pltpu.VMEM((CBR, CBW), _f32),
        pltpu.VMEM_SHARED((NCH, CBR, CBW), _f32),
        pltpu.VMEM_SHARED((NP,), _f32),
    ] + [pltpu.SemaphoreType.DMA] * 6,
)(_sc_edge_body)


def kernel(x, edge_index, batch, W1, a1_src, a1_dst, b1, W2, a2_src, a2_dst,
           b2, W_mu, b_mu, W_sigma, b_sigma):
    xt = jnp.pad(x, ((0, NP - N), (0, 0))).T
    batch_p = jnp.pad(batch, (0, NP - N), constant_values=NGRAPH)
    srcc = edge_index[0].reshape(2, NCH, CBR, CBW)
    dstc = edge_index[1].reshape(2, NCH, CBR, CBW)

    ht1, as1, ad1, mb1 = pl.pallas_call(
        _tc1_body,
        out_shape=[jax.ShapeDtypeStruct((D_H, NP), _f32),
                   jax.ShapeDtypeStruct((NP,), _f32),
                   jax.ShapeDtypeStruct((NP,), _f32),
                   jax.ShapeDtypeStruct((128,), _f32)],
    )(xt, W1.T, a1_src, a1_dst)

    osum1, den1 = _sc_edge(ht1, as1, ad1, mb1, srcc, dstc)

    ht2, as2, ad2, mb2 = pl.pallas_call(
        _tc2_body,
        out_shape=[jax.ShapeDtypeStruct((D_H, NP), _f32),
                   jax.ShapeDtypeStruct((NP,), _f32),
                   jax.ShapeDtypeStruct((NP,), _f32),
                   jax.ShapeDtypeStruct((128,), _f32)],
    )(osum1, den1, ht1, as1, ad1, b1, W2.T, a2_src, a2_dst)

    osum2, den2 = _sc_edge(ht2, as2, ad2, mb2, srcc, dstc)

    mu, sigma = pl.pallas_call(
        _tc3_body,
        out_shape=[jax.ShapeDtypeStruct((NGRAPH, SEQ_OUT * OUT_DIM), _f32),
                   jax.ShapeDtypeStruct((NGRAPH, SEQ_OUT * OUT_DIM), _f32)],
    )(osum2, den2, ht2, as2, ad2, b2, batch_p, W_mu, b_mu, W_sigma, b_sigma)

    return (mu.reshape(NGRAPH, SEQ_OUT, OUT_DIM),
            sigma.reshape(NGRAPH, SEQ_OUT, OUT_DIM))
